# Initial kernel scaffold; baseline (speedup 1.0000x reference)
import jax
import jax.numpy as jnp
from jax.experimental import pallas as pl


def kernel(output, mask, ind, target):
    raise NotImplementedError("write your pallas kernel here")


# speedup vs baseline: 3.8056x; 3.8056x over previous
import functools

import jax
import jax.numpy as jnp
from jax import lax
from jax.experimental import pallas as pl
from jax.experimental.pallas import tpu as pltpu
from jax.experimental.pallas import tpu_sc as plsc

L = 16
NW = 32


def _sc_partials(C, HW, K, table, indf, maskf, tgtt):
    P = (indf.shape[0]) // NW
    G = P // L
    NIDX = P * C
    NROW = NIDX // 128
    BPW = P // K

    mesh = plsc.VectorSubcoreMesh(core_axis_name="c", subcore_axis_name="s")

    @functools.partial(
        pl.kernel,
        out_type=(
            jax.ShapeDtypeStruct((NW, L), jnp.float32),
            jax.ShapeDtypeStruct((NW, L), jnp.float32),
        ),
        mesh=mesh,
        scratch_types=[
            pltpu.VMEM((P,), jnp.int32),
            pltpu.VMEM((P,), jnp.float32),
            pltpu.VMEM((P * C,), jnp.float32),
            pltpu.VMEM((NROW, 128), jnp.int32),
            pltpu.VMEM((NIDX,), jnp.float32),
            pltpu.VMEM((L,), jnp.float32),
            pltpu.VMEM((L,), jnp.float32),
            pltpu.SemaphoreType.DMA,
        ],
    )
    def k(table_h, ind_h, mask_h, tgt_h, oacc_h, omacc_h,
          ind_v, mask_v, tgt_v, idx_v, pbuf, acc_v, macc_v, sem):
        nc = 2
        wid = lax.axis_index("s") * nc + lax.axis_index("c")
        base = wid * P
        pltpu.sync_copy(ind_h.at[pl.ds(base, P)], ind_v)
        pltpu.sync_copy(mask_h.at[pl.ds(base, P)], mask_v)
        pltpu.sync_copy(tgt_h.at[pl.ds(base * C, P * C)], tgt_v)

        iota = lax.iota(jnp.int32, L)

        for g in range(G):
            ind_vec = ind_v[pl.ds(g * L, L)]
            pair_vec = base + g * L + iota
            b_vec = (pair_vec >> (K.bit_length() - 1)) * (C * HW)
            rowb = b_vec + ind_vec
            for c in range(C):
                q = g * (L * C) + c * L
                idx_v[q // 128, pl.ds(q % 128, L)] = rowb + c * HW

        cps = []
        for r in range(NROW):
            cps.append(pltpu.async_copy(
                table_h.at[idx_v.at[r]], pbuf.at[pl.ds(r * 128, 128)], sem))
        for cp in cps:
            cp.wait()

        def body(g, carry):
            acc, macc = carry
            mask_vec = mask_v[pl.ds(g * L, L)]
            gpb = g // (G // BPW)
            gk = g % (G // BPW)
            for c in range(C):
                pred = pbuf[pl.ds(g * (L * C) + c * L, L)]
                tgt = tgt_v[pl.ds(gpb * (C * K) + c * K + gk * L, L)]
                d = jnp.abs(pred - tgt)
                elem = jnp.where(d < 1.0, 0.5 * d * d, d - 0.5)
                acc = acc + elem * mask_vec
            macc = macc + mask_vec
            return acc, macc

        zero = jnp.zeros((L,), jnp.float32)
        acc, macc = lax.fori_loop(0, G, body, (zero, zero))
        acc_v[...] = acc
        macc_v[...] = macc
        pltpu.sync_copy(acc_v, oacc_h.at[wid])
        pltpu.sync_copy(macc_v, omacc_h.at[wid])

    return k(table, indf, maskf, tgtt)


def _combine(C, acc_ref, macc_ref, o_ref):
    s = jnp.sum(acc_ref[...])
    m = jnp.sum(macc_ref[...]) * C
    o_ref[...] = jnp.broadcast_to(s / m, (1, 1))


def kernel(output, mask, ind, target):
    B, C, H, W = output.shape
    K = ind.shape[1]

    table = output.reshape(-1)
    indf = ind.reshape(-1)
    maskf = mask.reshape(-1)
    tgtt = jnp.transpose(target, (0, 2, 1)).reshape(-1)

    oacc, omacc = _sc_partials(C, H * W, K, table, indf, maskf, tgtt)
    out = pl.pallas_call(
        functools.partial(_combine, float(C)),
        out_shape=jax.ShapeDtypeStruct((1, 1), jnp.float32),
    )(oacc, omacc)
    return out[0, 0]


# speedup vs baseline: 3.9112x; 1.0277x over previous
import functools

import jax
import jax.numpy as jnp
from jax import lax
from jax.experimental import pallas as pl
from jax.experimental.pallas import tpu as pltpu
from jax.experimental.pallas import tpu_sc as plsc

L = 16
NW = 32


def _sc_partials(C, HW, K, table, indf, maskf, tgtt):
    P = (indf.shape[0]) // NW
    G = P // L
    NIDX = P * C
    RPG = (L * C) // 128
    BPW = P // K
    KB = K.bit_length() - 1

    mesh = plsc.VectorSubcoreMesh(core_axis_name="c", subcore_axis_name="s")

    @functools.partial(
        pl.kernel,
        out_type=(
            jax.ShapeDtypeStruct((NW, L), jnp.float32),
            jax.ShapeDtypeStruct((NW, L), jnp.float32),
        ),
        mesh=mesh,
        scratch_types=[
            pltpu.VMEM((P,), jnp.int32),
            pltpu.VMEM((P,), jnp.float32),
            pltpu.VMEM((P * C,), jnp.float32),
            pltpu.VMEM((G * RPG, 128), jnp.int32),
            pltpu.VMEM((NIDX,), jnp.float32),
            pltpu.VMEM((L,), jnp.float32),
            pltpu.VMEM((L,), jnp.float32),
            pltpu.SemaphoreType.DMA,
            [pltpu.SemaphoreType.DMA] * G,
        ],
    )
    def k(table_h, ind_h, mask_h, tgt_h, oacc_h, omacc_h,
          ind_v, mask_v, tgt_v, idx_v, pbuf, acc_v, macc_v, sem_in, gsems):
        nc = 2
        wid = lax.axis_index("s") * nc + lax.axis_index("c")
        base = wid * P
        pltpu.sync_copy(ind_h.at[pl.ds(base, P)], ind_v)
        cp_t = pltpu.async_copy(
            tgt_h.at[pl.ds(base * C, P * C)], tgt_v, sem_in)
        cp_m = pltpu.async_copy(mask_h.at[pl.ds(base, P)], mask_v, sem_in)

        iota = lax.iota(jnp.int32, L)

        cps = []
        for g in range(G):
            ind_vec = ind_v[pl.ds(g * L, L)]
            pair_vec = base + g * L + iota
            rowb = (pair_vec >> KB) * (C * HW) + ind_vec
            for c in range(C):
                q = g * (L * C) + c * L
                idx_v[q // 128, pl.ds(q % 128, L)] = rowb + c * HW
            pair = []
            for r in range(RPG):
                row = g * RPG + r
                pair.append(pltpu.async_copy(
                    table_h.at[idx_v.at[row]],
                    pbuf.at[pl.ds(row * 128, 128)], gsems[g]))
            cps.append(pair)

        cp_t.wait()
        cp_m.wait()

        acc = jnp.zeros((L,), jnp.float32)
        macc = jnp.zeros((L,), jnp.float32)
        for g in range(G):
            for cp in cps[g]:
                cp.wait()
            mask_vec = mask_v[pl.ds(g * L, L)]
            gpb = g // (G // BPW)
            gk = g % (G // BPW)
            for c in range(C):
                pred = pbuf[pl.ds(g * (L * C) + c * L, L)]
                tgt = tgt_v[pl.ds(gpb * (C * K) + c * K + gk * L, L)]
                d = jnp.abs(pred - tgt)
                elem = jnp.where(d < 1.0, 0.5 * d * d, d - 0.5)
                acc = acc + elem * mask_vec
            macc = macc + mask_vec

        acc_v[...] = acc
        macc_v[...] = macc
        pltpu.sync_copy(acc_v, oacc_h.at[wid])
        pltpu.sync_copy(macc_v, omacc_h.at[wid])

    return k(table, indf, maskf, tgtt)


def _combine(C, acc_ref, macc_ref, o_ref):
    s = jnp.sum(acc_ref[...])
    m = jnp.sum(macc_ref[...]) * C
    o_ref[...] = jnp.broadcast_to(s / m, (1, 1))


def kernel(output, mask, ind, target):
    B, C, H, W = output.shape
    K = ind.shape[1]

    table = output.reshape(-1)
    indf = ind.reshape(-1)
    maskf = mask.reshape(-1)
    tgtt = jnp.transpose(target, (0, 2, 1)).reshape(-1)

    oacc, omacc = _sc_partials(C, H * W, K, table, indf, maskf, tgtt)
    out = pl.pallas_call(
        functools.partial(_combine, float(C)),
        out_shape=jax.ShapeDtypeStruct((1, 1), jnp.float32),
    )(oacc, omacc)
    return out[0, 0]
